# baseline (speedup 1.0000x reference)
import functools

import jax
import jax.numpy as jnp
from jax import lax
from jax.experimental import pallas as pl
from jax.experimental.pallas import tpu as pltpu
from jax.experimental.pallas import tpu_sc as plsc

N = 10000
E = 320000
D = 128
P = 10240

NC = 2
NS = 16
NW = NC * NS
EPW = E // NW

C = 80
NCH = EPW // C
CD = 80
NCHD = EPW // CD
RPS = P // NS
ZR = 128

_BLK = 1000
_G = N // _BLK

_sc_mesh = plsc.VectorSubcoreMesh(
    core_axis_name="c", subcore_axis_name="s", num_cores=NC, num_subcores=NS
)


@functools.partial(
    pl.kernel,
    out_type=jax.ShapeDtypeStruct((NC, P), jnp.float32),
    mesh=_sc_mesh,
    scratch_types=[
        pltpu.VMEM((NCHD, CD), jnp.int32),
        pltpu.VMEM((CD,), jnp.float32),
        pltpu.VMEM_SHARED((P,), jnp.float32),
        pltpu.SemaphoreType.DMA,
    ],
)
def _deg_sc(dst_hbm, out_hbm, didx, ones, dacc, sem):
    cid = lax.axis_index("c")
    sid = lax.axis_index("s")
    wid = sid * NC + cid

    pltpu.sync_copy(dst_hbm.at[wid], didx)

    for j in range(CD // 16):
        ones[pl.ds(j * 16, 16)] = jnp.zeros((16,), jnp.float32)
    for k in range(RPS // CD):
        pltpu.sync_copy(ones, dacc.at[pl.ds(sid * RPS + k * CD, CD)])
    for j in range(CD // 16):
        ones[pl.ds(j * 16, 16)] = jnp.full((16,), 1.0, jnp.float32)
    plsc.subcore_barrier()

    def body(k, carry):
        pltpu.async_copy(ones, dacc.at[didx.at[k]], sem, add=True)
        return carry

    lax.fori_loop(0, NCHD, body, 0)

    def drain(k, carry):
        pltpu.make_async_copy(ones, dacc.at[didx.at[0]], sem).wait()
        return carry

    lax.fori_loop(0, NCHD, drain, 0)
    plsc.subcore_barrier()
    sl = pl.ds(sid * RPS, RPS)
    pltpu.sync_copy(dacc.at[sl], out_hbm.at[cid, sl])


@functools.partial(
    pl.kernel,
    out_type=jax.ShapeDtypeStruct((NC, P, D), jnp.float32),
    mesh=_sc_mesh,
    scratch_types=[
        pltpu.VMEM((C,), jnp.int32),
        pltpu.VMEM((C,), jnp.int32),
        pltpu.VMEM((C,), jnp.int32),
        pltpu.VMEM((C,), jnp.int32),
        pltpu.VMEM((C,), jnp.int32),
        pltpu.VMEM((C,), jnp.int32),
        pltpu.VMEM((C, D), jnp.float32),
        pltpu.VMEM((C, D), jnp.float32),
        pltpu.VMEM((C, D), jnp.float32),
        pltpu.VMEM_SHARED((P, D), jnp.float32),
        pltpu.SemaphoreType.DMA,
        pltpu.SemaphoreType.DMA,
        pltpu.SemaphoreType.DMA,
        pltpu.SemaphoreType.DMA,
        pltpu.SemaphoreType.DMA,
        pltpu.SemaphoreType.DMA,
    ],
)
def _gs_sc(tab_hbm, src_hbm, dst_hbm, out_hbm,
           sa, da, sb, db, sc_, dc, ra, rb, rc, acc,
           ia, ib, ic, ga, gb, gc):
    cid = lax.axis_index("c")
    sid = lax.axis_index("s")
    wid = sid * NC + cid

    z16 = jnp.zeros((16,), jnp.float32)

    def zrow(i, carry):
        for j in range(D // 16):
            ra[i, pl.ds(j * 16, 16)] = z16
        return carry

    lax.fori_loop(0, C, zrow, 0)
    for k in range(RPS // C):
        pltpu.sync_copy(ra, acc.at[pl.ds(sid * RPS + k * C, C)])
    plsc.subcore_barrier()

    ebase = wid * EPW

    def ldidx(c, sx, dx, sem):
        off = ebase + c * C
        pltpu.async_copy(src_hbm.at[pl.ds(off, C)], sx, sem)
        pltpu.async_copy(dst_hbm.at[pl.ds(off, C)], dx, sem)

    def wtidx(sx, dx, sem):
        pltpu.make_async_copy(src_hbm.at[pl.ds(0, C)], sx, sem).wait()
        pltpu.make_async_copy(dst_hbm.at[pl.ds(0, C)], dx, sem).wait()

    def gwait(rx, gsem):
        pltpu.make_async_copy(tab_hbm.at[sa], rx, gsem).wait()

    ldidx(0, sa, da, ia)
    ldidx(1, sb, db, ib)
    ldidx(2, sc_, dc, ic)
    wtidx(sa, da, ia)
    pltpu.async_copy(tab_hbm.at[sa], ra, ga)
    wtidx(sb, db, ib)
    pltpu.async_copy(tab_hbm.at[sb], rb, gb)

    def step(c_next_idx, sx, dx, isem, rx, gsem, s_nx, d_nx, i_nx, r_nx, g_nx,
             c_gather, do_ld):
        wtidx(s_nx, d_nx, i_nx)
        pltpu.async_copy(tab_hbm.at[s_nx], r_nx, g_nx)
        gwait(rx, gsem)
        pltpu.sync_copy(rx, acc.at[dx], add=True)
        if do_ld is not None:
            @pl.when(do_ld)
            def _():
                ldidx(c_next_idx, sx, dx, isem)
        else:
            ldidx(c_next_idx, sx, dx, isem)

    def body(t, carry):
        c = 3 * t
        step(c + 3, sa, da, ia, ra, ga, sc_, dc, ic, rc, gc, c + 2, None)
        step(c + 4, sb, db, ib, rb, gb, sa, da, ia, ra, ga, c + 3, None)
        step(c + 5, sc_, dc, ic, rc, gc, sb, db, ib, rb, gb, c + 4,
             t < (NCH - 5) // 3)
        return carry

    lax.fori_loop(0, (NCH - 2) // 3, body, 0)
    gwait(ra, ga)
    pltpu.sync_copy(ra, acc.at[da], add=True)
    gwait(rb, gb)
    pltpu.sync_copy(rb, acc.at[db], add=True)

    plsc.subcore_barrier()
    for k in range(RPS // ZR):
        sl = pl.ds(sid * RPS + k * ZR, ZR)
        pltpu.sync_copy(acc.at[sl], out_hbm.at[cid, sl])


def _dinv(dp_ref):
    d = dp_ref[0] + dp_ref[1]
    return lax.rsqrt(jnp.maximum(d, 1.0))


def _prep_body(x_ref, w_ref, dp_ref, o_ref):
    o_ref[:] = jnp.dot(x_ref[:], w_ref[:],
                       preferred_element_type=jnp.float32) * _dinv(dp_ref)


_prep_tc = pl.pallas_call(
    _prep_body,
    grid=(_G,),
    in_specs=[
        pl.BlockSpec((_BLK, D), lambda i: (i, 0)),
        pl.BlockSpec((D, D), lambda i: (0, 0)),
        pl.BlockSpec((NC, _BLK, 1), lambda i: (0, i, 0)),
    ],
    out_specs=pl.BlockSpec((_BLK, D), lambda i: (i, 0)),
    out_shape=jax.ShapeDtypeStruct((N, D), jnp.float32),
)


def _mid_body(p_ref, dp_ref, b_ref, w_ref, o_ref):
    v = _dinv(dp_ref)
    h = jnp.maximum((p_ref[0] + p_ref[1]) * v + b_ref[:], 0.0)
    o_ref[:] = jnp.dot(h, w_ref[:], preferred_element_type=jnp.float32) * v


_mid_tc = pl.pallas_call(
    _mid_body,
    grid=(_G,),
    in_specs=[
        pl.BlockSpec((NC, _BLK, D), lambda i: (0, i, 0)),
        pl.BlockSpec((NC, _BLK, 1), lambda i: (0, i, 0)),
        pl.BlockSpec((1, D), lambda i: (0, 0)),
        pl.BlockSpec((D, D), lambda i: (0, 0)),
    ],
    out_specs=pl.BlockSpec((_BLK, D), lambda i: (i, 0)),
    out_shape=jax.ShapeDtypeStruct((N, D), jnp.float32),
)


def _fin_body(q_ref, dp_ref, b_ref, o_ref):
    o_ref[:] = (q_ref[0] + q_ref[1]) * _dinv(dp_ref) + b_ref[:]


_fin_tc = pl.pallas_call(
    _fin_body,
    grid=(_G,),
    in_specs=[
        pl.BlockSpec((NC, _BLK, D), lambda i: (0, i, 0)),
        pl.BlockSpec((NC, _BLK, 1), lambda i: (0, i, 0)),
        pl.BlockSpec((1, D), lambda i: (0, 0)),
    ],
    out_specs=pl.BlockSpec((_BLK, D), lambda i: (i, 0)),
    out_shape=jax.ShapeDtypeStruct((N, D), jnp.float32),
)


def kernel(x, edge_index, W1, b1, W2, b2):
    src = edge_index[0]
    dst = edge_index[1]
    dst3 = dst.reshape(NW, NCHD, CD)

    degp = _deg_sc(dst3).reshape(NC, P, 1)
    t1 = _prep_tc(x, W1, degp)
    p = _gs_sc(t1, src, dst)
    t2 = _mid_tc(p, degp, b1.reshape(1, D), W2)
    q = _gs_sc(t2, src, dst)
    return _fin_tc(q, degp, b2.reshape(1, D))


# speedup vs baseline: 1.0761x; 1.0761x over previous
import functools

import jax
import jax.numpy as jnp
from jax import lax
from jax.experimental import pallas as pl
from jax.experimental.pallas import tpu as pltpu
from jax.experimental.pallas import tpu_sc as plsc

N = 10000
E = 320000
D = 128
P = 10240

NC = 2
NS = 16
NW = NC * NS
EPW = E // NW

C = 128
NCH = (E // C) // NW
NREM = E // C - NCH * NW
EPW2 = NCH * C
RBASE = NW * EPW2
CD = 80
NCHD = EPW // CD
RPS = P // NS
ZR = 128

_BLK = 1000
_G = N // _BLK

_sc_mesh = plsc.VectorSubcoreMesh(
    core_axis_name="c", subcore_axis_name="s", num_cores=NC, num_subcores=NS
)


@functools.partial(
    pl.kernel,
    out_type=jax.ShapeDtypeStruct((NC, P), jnp.float32),
    mesh=_sc_mesh,
    scratch_types=[
        pltpu.VMEM((NCHD, CD), jnp.int32),
        pltpu.VMEM((CD,), jnp.float32),
        pltpu.VMEM_SHARED((P,), jnp.float32),
        pltpu.SemaphoreType.DMA,
    ],
)
def _deg_sc(dst_hbm, out_hbm, didx, ones, dacc, sem):
    cid = lax.axis_index("c")
    sid = lax.axis_index("s")
    wid = sid * NC + cid

    pltpu.sync_copy(dst_hbm.at[wid], didx)

    for j in range(CD // 16):
        ones[pl.ds(j * 16, 16)] = jnp.zeros((16,), jnp.float32)
    for k in range(RPS // CD):
        pltpu.sync_copy(ones, dacc.at[pl.ds(sid * RPS + k * CD, CD)])
    for j in range(CD // 16):
        ones[pl.ds(j * 16, 16)] = jnp.full((16,), 1.0, jnp.float32)
    plsc.subcore_barrier()

    def body(k, carry):
        pltpu.async_copy(ones, dacc.at[didx.at[k]], sem, add=True)
        return carry

    lax.fori_loop(0, NCHD, body, 0)

    def drain(k, carry):
        pltpu.make_async_copy(ones, dacc.at[didx.at[0]], sem).wait()
        return carry

    lax.fori_loop(0, NCHD, drain, 0)
    plsc.subcore_barrier()
    sl = pl.ds(sid * RPS, RPS)
    pltpu.sync_copy(dacc.at[sl], out_hbm.at[cid, sl])


@functools.partial(
    pl.kernel,
    out_type=jax.ShapeDtypeStruct((NC, P, D), jnp.float32),
    mesh=_sc_mesh,
    scratch_types=[
        pltpu.VMEM((C,), jnp.int32),
        pltpu.VMEM((C,), jnp.int32),
        pltpu.VMEM((C,), jnp.int32),
        pltpu.VMEM((C,), jnp.int32),
        pltpu.VMEM((C, D), jnp.float32),
        pltpu.VMEM((C, D), jnp.float32),
        pltpu.VMEM_SHARED((P, D), jnp.float32),
        pltpu.SemaphoreType.DMA,
        pltpu.SemaphoreType.DMA,
        pltpu.SemaphoreType.DMA,
        pltpu.SemaphoreType.DMA,
    ],
)
def _gs_sc(tab_hbm, src_hbm, dst_hbm, out_hbm, sidx0, didx0, sidx1, didx1,
           rows0, rows1, acc, semi0, semi1, semg0, semg1):
    cid = lax.axis_index("c")
    sid = lax.axis_index("s")
    wid = sid * NC + cid

    z16 = jnp.zeros((16,), jnp.float32)

    def zrow(i, carry):
        for j in range(D // 16):
            rows0[i, pl.ds(j * 16, 16)] = z16
        return carry

    ebase = wid * EPW2

    def ldidx(off, sb, db, sem):
        pltpu.async_copy(src_hbm.at[pl.ds(off, C)], sb, sem)
        pltpu.async_copy(dst_hbm.at[pl.ds(off, C)], db, sem)

    def wtidx(sb, db, sem):
        pltpu.make_async_copy(src_hbm.at[pl.ds(0, C)], sb, sem).wait()
        pltpu.make_async_copy(dst_hbm.at[pl.ds(0, C)], db, sem).wait()

    ldidx(ebase, sidx0, didx0, semi0)
    ldidx(ebase + C, sidx1, didx1, semi1)

    lax.fori_loop(0, C, zrow, 0)
    for k in range(RPS // C):
        pltpu.async_copy(rows0, acc.at[pl.ds(sid * RPS + k * C, C)], semg1)
    for k in range(RPS // C):
        pltpu.make_async_copy(rows0, acc.at[pl.ds(0, C)], semg1).wait()
    plsc.subcore_barrier()

    NT = NCH // 2
    wtidx(sidx0, didx0, semi0)
    pltpu.async_copy(tab_hbm.at[sidx0], rows0, semg0)

    def body(t, carry):
        wtidx(sidx1, didx1, semi1)
        pltpu.async_copy(tab_hbm.at[sidx1], rows1, semg1)
        pltpu.make_async_copy(tab_hbm.at[sidx0], rows0, semg0).wait()
        pltpu.sync_copy(rows0, acc.at[didx0], add=True)

        @pl.when(t < NT - 1)
        def _():
            ldidx(ebase + (2 * t + 2) * C, sidx0, didx0, semi0)
            wtidx(sidx0, didx0, semi0)
            pltpu.async_copy(tab_hbm.at[sidx0], rows0, semg0)

        pltpu.make_async_copy(tab_hbm.at[sidx1], rows1, semg1).wait()
        pltpu.sync_copy(rows1, acc.at[didx1], add=True)

        @pl.when(t < NT - 1)
        def _():
            ldidx(ebase + (2 * t + 3) * C, sidx1, didx1, semi1)

        return carry

    lax.fori_loop(0, NT, body, 0)

    @pl.when(wid < NREM)
    def _():
        ldidx(RBASE + wid * C, sidx0, didx0, semi0)
        wtidx(sidx0, didx0, semi0)
        pltpu.async_copy(tab_hbm.at[sidx0], rows0, semg0)
        pltpu.make_async_copy(tab_hbm.at[sidx0], rows0, semg0).wait()
        pltpu.sync_copy(rows0, acc.at[didx0], add=True)

    plsc.subcore_barrier()
    for k in range(RPS // ZR):
        sl = pl.ds(sid * RPS + k * ZR, ZR)
        pltpu.sync_copy(acc.at[sl], out_hbm.at[cid, sl])


def _dinv(dp_ref):
    d = dp_ref[0] + dp_ref[1]
    return lax.rsqrt(jnp.maximum(d, 1.0))


def _prep_body(x_ref, w_ref, dp_ref, o_ref):
    o_ref[:] = jnp.dot(x_ref[:], w_ref[:],
                       preferred_element_type=jnp.float32) * _dinv(dp_ref)


_prep_tc = pl.pallas_call(
    _prep_body,
    grid=(_G,),
    in_specs=[
        pl.BlockSpec((_BLK, D), lambda i: (i, 0)),
        pl.BlockSpec((D, D), lambda i: (0, 0)),
        pl.BlockSpec((NC, _BLK, 1), lambda i: (0, i, 0)),
    ],
    out_specs=pl.BlockSpec((_BLK, D), lambda i: (i, 0)),
    out_shape=jax.ShapeDtypeStruct((N, D), jnp.float32),
)


def _mid_body(p_ref, dp_ref, b_ref, w_ref, o_ref):
    v = _dinv(dp_ref)
    h = jnp.maximum((p_ref[0] + p_ref[1]) * v + b_ref[:], 0.0)
    o_ref[:] = jnp.dot(h, w_ref[:], preferred_element_type=jnp.float32) * v


_mid_tc = pl.pallas_call(
    _mid_body,
    grid=(_G,),
    in_specs=[
        pl.BlockSpec((NC, _BLK, D), lambda i: (0, i, 0)),
        pl.BlockSpec((NC, _BLK, 1), lambda i: (0, i, 0)),
        pl.BlockSpec((1, D), lambda i: (0, 0)),
        pl.BlockSpec((D, D), lambda i: (0, 0)),
    ],
    out_specs=pl.BlockSpec((_BLK, D), lambda i: (i, 0)),
    out_shape=jax.ShapeDtypeStruct((N, D), jnp.float32),
)


def _fin_body(q_ref, dp_ref, b_ref, o_ref):
    o_ref[:] = (q_ref[0] + q_ref[1]) * _dinv(dp_ref) + b_ref[:]


_fin_tc = pl.pallas_call(
    _fin_body,
    grid=(_G,),
    in_specs=[
        pl.BlockSpec((NC, _BLK, D), lambda i: (0, i, 0)),
        pl.BlockSpec((NC, _BLK, 1), lambda i: (0, i, 0)),
        pl.BlockSpec((1, D), lambda i: (0, 0)),
    ],
    out_specs=pl.BlockSpec((_BLK, D), lambda i: (i, 0)),
    out_shape=jax.ShapeDtypeStruct((N, D), jnp.float32),
)


def kernel(x, edge_index, W1, b1, W2, b2):
    src = edge_index[0]
    dst = edge_index[1]
    dst3 = dst.reshape(NW, NCHD, CD)

    degp = _deg_sc(dst3).reshape(NC, P, 1)
    t1 = _prep_tc(x, W1, degp)
    p = _gs_sc(t1, src, dst)
    t2 = _mid_tc(p, degp, b1.reshape(1, D), W2)
    q = _gs_sc(t2, src, dst)
    return _fin_tc(q, degp, b2.reshape(1, D))


# speedup vs baseline: 1.1975x; 1.1128x over previous
import functools

import jax
import jax.numpy as jnp
from jax import lax
from jax.experimental import pallas as pl
from jax.experimental.pallas import tpu as pltpu
from jax.experimental.pallas import tpu_sc as plsc

N = 10000
E = 320000
D = 128
P = 10240

NC = 2
NS = 16
NW = NC * NS
EPW = E // NW

C = 128
NCH = (E // C) // NW
NREM = E // C - NCH * NW
EPW2 = NCH * C
RBASE = NW * EPW2
CD = 80
NCHD = EPW // CD
RPS = P // NS
ZR = 128

_BLK = 1000
_G = N // _BLK

_sc_mesh = plsc.VectorSubcoreMesh(
    core_axis_name="c", subcore_axis_name="s", num_cores=NC, num_subcores=NS
)


@functools.partial(
    pl.kernel,
    out_type=jax.ShapeDtypeStruct((NC, P), jnp.float32),
    mesh=_sc_mesh,
    scratch_types=[
        pltpu.VMEM((NCHD, CD), jnp.int32),
        pltpu.VMEM((CD,), jnp.float32),
        pltpu.VMEM_SHARED((P,), jnp.float32),
        pltpu.SemaphoreType.DMA,
    ],
)
def _deg_sc(dst_hbm, out_hbm, didx, ones, dacc, sem):
    cid = lax.axis_index("c")
    sid = lax.axis_index("s")
    wid = sid * NC + cid

    pltpu.sync_copy(dst_hbm.at[wid], didx)

    for j in range(CD // 16):
        ones[pl.ds(j * 16, 16)] = jnp.zeros((16,), jnp.float32)
    for k in range(RPS // CD):
        pltpu.sync_copy(ones, dacc.at[pl.ds(sid * RPS + k * CD, CD)])
    for j in range(CD // 16):
        ones[pl.ds(j * 16, 16)] = jnp.full((16,), 1.0, jnp.float32)
    plsc.subcore_barrier()

    def body(k, carry):
        pltpu.async_copy(ones, dacc.at[didx.at[k]], sem, add=True)
        return carry

    lax.fori_loop(0, NCHD, body, 0)

    def drain(k, carry):
        pltpu.make_async_copy(ones, dacc.at[didx.at[0]], sem).wait()
        return carry

    lax.fori_loop(0, NCHD, drain, 0)
    plsc.subcore_barrier()
    sl = pl.ds(sid * RPS, RPS)
    pltpu.sync_copy(dacc.at[sl], out_hbm.at[cid, sl])


@functools.partial(
    pl.kernel,
    out_type=jax.ShapeDtypeStruct((NC, P, D), jnp.float32),
    mesh=_sc_mesh,
    scratch_types=[
        pltpu.VMEM((C,), jnp.int32),
        pltpu.VMEM((C,), jnp.int32),
        pltpu.VMEM((C,), jnp.int32),
        pltpu.VMEM((C,), jnp.int32),
        pltpu.VMEM((C,), jnp.int32),
        pltpu.VMEM((C,), jnp.int32),
        pltpu.VMEM((C,), jnp.int32),
        pltpu.VMEM((C,), jnp.int32),
        pltpu.VMEM((C, D), jnp.float32),
        pltpu.VMEM((C, D), jnp.float32),
        pltpu.VMEM_SHARED((P, D), jnp.float32),
        pltpu.SemaphoreType.DMA,
        pltpu.SemaphoreType.DMA,
        pltpu.SemaphoreType.DMA,
        pltpu.SemaphoreType.DMA,
        pltpu.SemaphoreType.DMA,
        pltpu.SemaphoreType.DMA,
        pltpu.SemaphoreType.DMA,
        pltpu.SemaphoreType.DMA,
    ],
)
def _gs_sc(tab_hbm, src_hbm, dst_hbm, out_hbm,
           s0, d0, s1, d1, s2, d2, s3, d3, r0, r1, acc,
           i0, i1, i2, i3, g0, g1, ss0, ss1):
    cid = lax.axis_index("c")
    sid = lax.axis_index("s")
    wid = sid * NC + cid
    SX, DX, IX = (s0, s1, s2, s3), (d0, d1, d2, d3), (i0, i1, i2, i3)
    RX, GX, SS = (r0, r1), (g0, g1), (ss0, ss1)

    ebase = wid * EPW2

    def ldidx(off, q):
        pltpu.async_copy(src_hbm.at[pl.ds(off, C)], SX[q], IX[q])
        pltpu.async_copy(dst_hbm.at[pl.ds(off, C)], DX[q], IX[q])

    def wtidx(q):
        pltpu.make_async_copy(src_hbm.at[pl.ds(0, C)], SX[q], IX[q]).wait()
        pltpu.make_async_copy(dst_hbm.at[pl.ds(0, C)], DX[q], IX[q]).wait()

    def gwait(p):
        pltpu.make_async_copy(tab_hbm.at[s0], RX[p], GX[p]).wait()

    def scwait(p):
        pltpu.make_async_copy(RX[p], acc.at[d0], SS[p]).wait()

    ldidx(ebase, 0)
    ldidx(ebase + C, 1)
    ldidx(ebase + 2 * C, 2)

    z16 = jnp.zeros((16,), jnp.float32)

    def zrow(i, carry):
        for j in range(D // 16):
            r0[i, pl.ds(j * 16, 16)] = z16
        return carry

    lax.fori_loop(0, C, zrow, 0)
    for k in range(RPS // C):
        pltpu.async_copy(r0, acc.at[pl.ds(sid * RPS + k * C, C)], g1)
    for k in range(RPS // C):
        pltpu.make_async_copy(r0, acc.at[pl.ds(0, C)], g1).wait()
    plsc.subcore_barrier()

    wtidx(0)
    pltpu.async_copy(tab_hbm.at[s0], r0, g0)

    def substep(cg, p, q, first, do_ld):
        q1 = (q + 1) % 4
        wtidx(q1)
        if not first:
            scwait(1 - p)
        pltpu.async_copy(tab_hbm.at[SX[q1]], RX[1 - p], GX[1 - p])
        gwait(p)
        pltpu.async_copy(RX[p], acc.at[DX[q]], SS[p], add=True)
        if do_ld is not None:
            off = ebase + (cg + 2) * C
            if do_ld is True:
                ldidx(off, (q + 3) % 4)
            else:
                @pl.when(do_ld)
                def _():
                    ldidx(off, (q + 3) % 4)

    def body(t, carry):
        c = 4 * t
        substep(c + 1, 0, 0, False, True)
        substep(c + 2, 1, 1, False, True)
        substep(c + 3, 0, 2, False, True)
        substep(c + 4, 1, 3, False, t < (NCH - 4) // 4)
        return carry

    substep(1, 0, 0, True, True)
    substep(2, 1, 1, False, True)
    substep(3, 0, 2, False, True)
    substep(4, 1, 3, False, True)

    def body_shifted(t, carry):
        return body(t + 1, carry)

    lax.fori_loop(0, (NCH - 4) // 4, body_shifted, 0)
    wtidx(1)
    scwait(1)
    pltpu.async_copy(tab_hbm.at[SX[1]], RX[1], GX[1])
    gwait(0)
    pltpu.async_copy(RX[0], acc.at[DX[0]], SS[0], add=True)
    gwait(1)
    pltpu.async_copy(RX[1], acc.at[DX[1]], SS[1], add=True)
    scwait(0)
    scwait(1)

    @pl.when(wid < NREM)
    def _():
        ldidx(RBASE + wid * C, 2)
        wtidx(2)
        pltpu.async_copy(tab_hbm.at[s2], r0, g0)
        gwait(0)
        pltpu.sync_copy(r0, acc.at[d2], add=True)

    plsc.subcore_barrier()
    for k in range(RPS // ZR):
        sl = pl.ds(sid * RPS + k * ZR, ZR)
        pltpu.sync_copy(acc.at[sl], out_hbm.at[cid, sl])


def _dinv(dp_ref):
    d = dp_ref[0] + dp_ref[1]
    return lax.rsqrt(jnp.maximum(d, 1.0))


def _prep_body(x_ref, w_ref, dp_ref, o_ref):
    o_ref[:] = jnp.dot(x_ref[:], w_ref[:],
                       preferred_element_type=jnp.float32) * _dinv(dp_ref)


_prep_tc = pl.pallas_call(
    _prep_body,
    grid=(_G,),
    in_specs=[
        pl.BlockSpec((_BLK, D), lambda i: (i, 0)),
        pl.BlockSpec((D, D), lambda i: (0, 0)),
        pl.BlockSpec((NC, _BLK, 1), lambda i: (0, i, 0)),
    ],
    out_specs=pl.BlockSpec((_BLK, D), lambda i: (i, 0)),
    out_shape=jax.ShapeDtypeStruct((N, D), jnp.float32),
)


def _mid_body(p_ref, dp_ref, b_ref, w_ref, o_ref):
    v = _dinv(dp_ref)
    h = jnp.maximum((p_ref[0] + p_ref[1]) * v + b_ref[:], 0.0)
    o_ref[:] = jnp.dot(h, w_ref[:], preferred_element_type=jnp.float32) * v


_mid_tc = pl.pallas_call(
    _mid_body,
    grid=(_G,),
    in_specs=[
        pl.BlockSpec((NC, _BLK, D), lambda i: (0, i, 0)),
        pl.BlockSpec((NC, _BLK, 1), lambda i: (0, i, 0)),
        pl.BlockSpec((1, D), lambda i: (0, 0)),
        pl.BlockSpec((D, D), lambda i: (0, 0)),
    ],
    out_specs=pl.BlockSpec((_BLK, D), lambda i: (i, 0)),
    out_shape=jax.ShapeDtypeStruct((N, D), jnp.float32),
)


def _fin_body(q_ref, dp_ref, b_ref, o_ref):
    o_ref[:] = (q_ref[0] + q_ref[1]) * _dinv(dp_ref) + b_ref[:]


_fin_tc = pl.pallas_call(
    _fin_body,
    grid=(_G,),
    in_specs=[
        pl.BlockSpec((NC, _BLK, D), lambda i: (0, i, 0)),
        pl.BlockSpec((NC, _BLK, 1), lambda i: (0, i, 0)),
        pl.BlockSpec((1, D), lambda i: (0, 0)),
    ],
    out_specs=pl.BlockSpec((_BLK, D), lambda i: (i, 0)),
    out_shape=jax.ShapeDtypeStruct((N, D), jnp.float32),
)


def kernel(x, edge_index, W1, b1, W2, b2):
    src = edge_index[0]
    dst = edge_index[1]
    dst3 = dst.reshape(NW, NCHD, CD)

    degp = _deg_sc(dst3).reshape(NC, P, 1)
    t1 = _prep_tc(x, W1, degp)
    p = _gs_sc(t1, src, dst)
    t2 = _mid_tc(p, degp, b1.reshape(1, D), W2)
    q = _gs_sc(t2, src, dst)
    return _fin_tc(q, degp, b2.reshape(1, D))
